# Initial kernel scaffold; baseline (speedup 1.0000x reference)
import jax
import jax.numpy as jnp
from jax.experimental import pallas as pl


def kernel(x, edge_index, edge_attr, batch, W_init, b_init, We, be, W1, b1, W2, b2, W_ffn, b_ffn):
    raise NotImplementedError("write your pallas kernel here")


# speedup vs baseline: 3.6596x; 3.6596x over previous
import functools

import jax
import jax.numpy as jnp
from jax import lax
from jax.experimental import pallas as pl
from jax.experimental.pallas import tpu as pltpu
from jax.experimental.pallas import tpu_sc as plsc

_N = 10000
_E = 320000
_D = 128
_DE = 16
_DEPTH = 3
_NG = 64

_NC = 2
_NS = 16
_NW = _NC * _NS
_EPW = _E // _NW
_CH = 80
_NCHUNK = _EPW // _CH
_RPT = 624
_ZR = 48
_TAIL = _N - _NS * _RPT


def _init_body(x_ref, w_ref, b_ref, o_ref):
    o_ref[...] = jnp.maximum(
        jnp.dot(x_ref[...], w_ref[...], preferred_element_type=jnp.float32)
        + b_ref[...], 0.0)


def _tc_init(x, W, b2d):
    return pl.pallas_call(
        _init_body,
        grid=(10,),
        in_specs=[pl.BlockSpec((1000, _D), lambda i: (i, 0)),
                  pl.BlockSpec((_D, _D), lambda i: (0, 0)),
                  pl.BlockSpec((1, _D), lambda i: (0, 0))],
        out_specs=pl.BlockSpec((1000, _D), lambda i: (i, 0)),
        out_shape=jax.ShapeDtypeStruct((_N, _D), jnp.float32),
    )(x, W, b2d)


def _edge_body(a_ref, w_ref, b_ref, o_ref):
    o_ref[...] = (
        jnp.dot(a_ref[...], w_ref[...], preferred_element_type=jnp.float32)
        + b_ref[...])


def _tc_edge(attr, Wl, bl2d):
    return pl.pallas_call(
        _edge_body,
        grid=(40,),
        in_specs=[pl.BlockSpec((8000, _DE), lambda i: (i, 0)),
                  pl.BlockSpec((_DE, _D), lambda i: (0, 0)),
                  pl.BlockSpec((1, _D), lambda i: (0, 0))],
        out_specs=pl.BlockSpec((8000, _D), lambda i: (i, 0)),
        out_shape=jax.ShapeDtypeStruct((_E, _D), jnp.float32),
    )(attr, Wl, bl2d)


def _node_body(h_ref, a0_ref, a1_ref, h0_ref, w1_ref, b1_ref, w2_ref, b2_ref,
               o_ref):
    z = h_ref[...] + a0_ref[0] + a1_ref[0]
    t = jnp.maximum(
        jnp.dot(z, w1_ref[...], preferred_element_type=jnp.float32)
        + b1_ref[...], 0.0)
    o_ref[...] = jnp.maximum(
        jnp.dot(t, w2_ref[...], preferred_element_type=jnp.float32)
        + b2_ref[...] + h0_ref[...], 0.0)


def _tc_node(h, agg, h0, W1l, b1l2d, W2l, b2l2d):
    return pl.pallas_call(
        _node_body,
        grid=(10,),
        in_specs=[pl.BlockSpec((1000, _D), lambda i: (i, 0)),
                  pl.BlockSpec((1, 1000, _D), lambda i: (0, i, 0)),
                  pl.BlockSpec((1, 1000, _D), lambda i: (1, i, 0)),
                  pl.BlockSpec((1000, _D), lambda i: (i, 0)),
                  pl.BlockSpec((_D, _D), lambda i: (0, 0)),
                  pl.BlockSpec((1, _D), lambda i: (0, 0)),
                  pl.BlockSpec((_D, _D), lambda i: (0, 0)),
                  pl.BlockSpec((1, _D), lambda i: (0, 0))],
        out_specs=pl.BlockSpec((1000, _D), lambda i: (i, 0)),
        out_shape=jax.ShapeDtypeStruct((_N, _D), jnp.float32),
    )(h, agg, agg, h0, W1l, b1l2d, W2l, b2l2d)


def _final_body(h_ref, b_ref, wf_ref, bf_ref, o_ref):
    oneh = (b_ref[...] == lax.broadcasted_iota(jnp.int32, (_NG, _N), 0)
            ).astype(jnp.float32)
    pooled = jnp.dot(oneh, h_ref[...], preferred_element_type=jnp.float32)
    o_ref[...] = (
        jnp.dot(pooled, wf_ref[...], preferred_element_type=jnp.float32)
        + bf_ref[...])


def _tc_final(h, batch2d, Wf, bf2d):
    return pl.pallas_call(
        _final_body,
        out_shape=jax.ShapeDtypeStruct((_NG, 1), jnp.float32),
    )(h, batch2d, Wf, bf2d)


def _sc_edge_pass(h, e, edge_index):
    mesh = plsc.VectorSubcoreMesh(core_axis_name="c", subcore_axis_name="s")

    @functools.partial(
        pl.kernel,
        out_type=jax.ShapeDtypeStruct((_NC, _N, _D), jnp.float32),
        mesh=mesh,
        scratch_types=[
            pltpu.VMEM((_EPW,), jnp.int32),
            pltpu.VMEM((_EPW,), jnp.int32),
            pltpu.VMEM((_CH,), jnp.int32),
            pltpu.VMEM((_CH, _D), jnp.float32),
            pltpu.VMEM((_CH, _D), jnp.float32),
            pltpu.VMEM_SHARED((_N, _D), jnp.float32),
            pltpu.SemaphoreType.DMA,
            pltpu.SemaphoreType.DMA,
        ],
    )
    def k(h_hbm, e_hbm, ei_hbm, out_hbm,
          src_v, dst_v, dstc_v, rows_v, ev_v, agg_sh, sem_g, sem_e):
        c = lax.axis_index("c")
        s = lax.axis_index("s")
        wid = c * _NS + s
        base0 = wid * _EPW
        row0 = s * _RPT

        def _zb(r, carry):
            for q in range(_D // 16):
                rows_v[r, pl.ds(q * 16, 16)] = jnp.zeros((16,), jnp.float32)
            return carry
        lax.fori_loop(0, _CH, _zb, 0)
        for z in range(_RPT // _ZR):
            pltpu.sync_copy(rows_v.at[pl.ds(0, _ZR)],
                            agg_sh.at[pl.ds(row0 + z * _ZR, _ZR)])

        @pl.when(s == _NS - 1)
        def _zero_tail():
            pltpu.sync_copy(rows_v.at[pl.ds(0, _TAIL)],
                            agg_sh.at[pl.ds(_NS * _RPT, _TAIL)])
        plsc.subcore_barrier()

        pltpu.sync_copy(ei_hbm.at[pl.ds(base0, _EPW)], src_v)
        pltpu.sync_copy(ei_hbm.at[pl.ds(_E + base0, _EPW)], dst_v)

        def _chunk(i, carry):
            off = i * _CH
            for q in range(_CH // 16):
                dstc_v[pl.ds(q * 16, 16)] = dst_v[pl.ds(off + q * 16, 16)]
            cg = pltpu.async_copy(h_hbm.at[src_v.at[pl.ds(off, _CH)]],
                                  rows_v, sem_g)
            ce = pltpu.async_copy(e_hbm.at[pl.ds(base0 + off, _CH)],
                                  ev_v, sem_e)
            cg.wait()
            ce.wait()

            def _row(r, rcarry):
                for q in range(_D // 16):
                    sl = pl.ds(q * 16, 16)
                    rows_v[r, sl] = jnp.maximum(rows_v[r, sl] + ev_v[r, sl],
                                                0.0)
                return rcarry
            lax.fori_loop(0, _CH, _row, 0)
            pltpu.sync_copy(rows_v, agg_sh.at[dstc_v], add=True)
            return carry
        lax.fori_loop(0, _NCHUNK, _chunk, 0)
        plsc.subcore_barrier()

        for z in range(_RPT // _ZR):
            r0 = row0 + z * _ZR
            pltpu.sync_copy(agg_sh.at[pl.ds(r0, _ZR)], rows_v.at[pl.ds(0, _ZR)])
            pltpu.sync_copy(rows_v.at[pl.ds(0, _ZR)], out_hbm.at[c, pl.ds(r0, _ZR)])

        @pl.when(s == _NS - 1)
        def _copy_tail():
            t0 = _NS * _RPT
            pltpu.sync_copy(agg_sh.at[pl.ds(t0, _TAIL)],
                            rows_v.at[pl.ds(0, _TAIL)])
            pltpu.sync_copy(rows_v.at[pl.ds(0, _TAIL)],
                            out_hbm.at[c, pl.ds(t0, _TAIL)])

    return k(h, e, edge_index.reshape(2 * _E))


def kernel(x, edge_index, edge_attr, batch, W_init, b_init, We, be,
           W1, b1, W2, b2, W_ffn, b_ffn):
    h = _tc_init(x, W_init, b_init.reshape(1, _D))
    h0 = h
    for l in range(_DEPTH):
        e = _tc_edge(edge_attr, We[l], be[l].reshape(1, _D))
        agg = _sc_edge_pass(h, e, edge_index)
        h = _tc_node(h, agg, h0, W1[l], b1[l].reshape(1, _D),
                     W2[l], b2[l].reshape(1, _D))
    out2 = _tc_final(h, batch.reshape(1, _N), W_ffn, b_ffn.reshape(1, 1))
    return out2[:, 0]
